# baseline (speedup 1.0000x reference)
import functools

import jax
import jax.numpy as jnp
from jax import lax
from jax.experimental import pallas as pl
from jax.experimental.pallas import tpu as pltpu
from jax.experimental.pallas import tpu_sc as plsc

N = 100000
NH = N // 2
E = 1600000
NS = 16
L = 16

K = 400
SB = 2000
EPS = 1e-16
CHUNKS = E // NS // K
STAGE = 10000
MIDR = 5000

_mesh = plsc.VectorSubcoreMesh(core_axis_name="c", subcore_axis_name="s")
_SC_PARAMS = pltpu.CompilerParams(needs_layout_passes=False,
                                  use_tc_tiling_on_sc=False)


def _sc_fused(epk_h, h_hbm, als_h, ald_h, rec_h, b1_h, wt_h,
              acc2i_o, ald2_o,
              acc_sp, s_sp, acc2_sp,
              ek_a, als_a, ald_a, hrow_a, msg_a, p_a, dl_a,
              ek_b, als_b, ald_b, hrow_b, msg_b, p_b, dl_b,
              msg2_a, msg2_b, c_b, sb, b1_v, wt_v,
              sem_e, sem_g, sem_s):
    cid = lax.axis_index("c")
    sid = lax.axis_index("s")
    is_hi = cid == 1

    A = (ek_a, als_a, ald_a, hrow_a, msg_a, p_a, dl_a, msg2_a)
    B = (ek_b, als_b, ald_b, hrow_b, msg_b, p_b, dl_b, msg2_b)

    zv = jnp.zeros((L,), jnp.float32)

    def _zm(j, _):
        msg_a[j] = zv
        return ()

    lax.fori_loop(0, K, _zm, (), unroll=8)

    def _zs(j, _):
        sb[pl.ds(j * L, L)] = zv
        return ()

    lax.fori_loop(0, SB // L, _zs, (), unroll=8)

    @pl.when(sid < 10)
    def _zero_acc():
        for t in range(MIDR // 200):
            rs = pl.ds(sid * MIDR + t * 200, 200)
            pltpu.sync_copy(msg_a.at[pl.ds(0, 200)], acc_sp.at[rs])

    @pl.when(sid < 5)
    def _zero_s():
        for t in range(STAGE // SB):
            rs = pl.ds(sid * STAGE + t * SB, SB)
            pltpu.sync_copy(sb, s_sp.at[rs])

    riota = lax.iota(jnp.int32, L) >> 3
    ciota = lax.iota(jnp.int32, L) & 7
    zv16 = jnp.zeros((L,), jnp.float32)

    def _zm2(j, _):
        rr = riota + 2 * j
        plsc.store_scatter(msg2_a, [rr, ciota], zv16)
        plsc.store_scatter(msg2_b, [rr, ciota], zv16)
        return ()

    lax.fori_loop(0, K // 2, _zm2, (), unroll=8)

    @pl.when(sid < 10)
    def _zero_acc2():
        for t in range(MIDR // 200):
            rs = pl.ds(sid * MIDR + t * 200, 200)
            pltpu.sync_copy(msg2_a.at[pl.ds(0, 200)], acc2_sp.at[rs])

    pltpu.sync_copy(b1_h, b1_v)
    pltpu.sync_copy(wt_h, wt_v)

    plsc.subcore_barrier()

    cbase = sid * CHUNKS

    def _issue_eb(bufs, k):
        pltpu.async_copy(epk_h.at[cbase + k], bufs[0], sem_e)

    def _wait_eb(bufs, k):
        pltpu.make_async_copy(epk_h.at[cbase + k], bufs[0], sem_e).wait()

    def _issue_g1(bufs):
        ekr = bufs[0]
        pltpu.async_copy(als_h.at[ekr.at[pl.ds(0, K)]], bufs[1], sem_g)
        pltpu.async_copy(ald_h.at[ekr.at[pl.ds(K, K)]], bufs[2], sem_g)
        pltpu.async_copy(h_hbm.at[ekr.at[pl.ds(0, K)]], bufs[3], sem_g)

    def _wait_g1(bufs):
        ekr = bufs[0]
        pltpu.make_async_copy(als_h.at[ekr.at[pl.ds(0, K)]], bufs[1],
                              sem_g).wait()
        pltpu.make_async_copy(ald_h.at[ekr.at[pl.ds(K, K)]], bufs[2],
                              sem_g).wait()
        pltpu.make_async_copy(h_hbm.at[ekr.at[pl.ds(0, K)]], bufs[3],
                              sem_g).wait()

    def _issue_sc1(bufs):
        pltpu.async_copy(bufs[4], acc_sp.at[bufs[6]], sem_s, add=True)
        pltpu.async_copy(bufs[5], s_sp.at[bufs[6]], sem_s, add=True)

    def _wait_sc1(bufs):
        pltpu.make_async_copy(bufs[4], acc_sp.at[bufs[6]], sem_s).wait()
        pltpu.make_async_copy(bufs[5], s_sp.at[bufs[6]], sem_s).wait()

    def _vec1(bufs):
        (ekr, alsr, aldr, _, _, pr, dlr, _) = bufs

        def vec(g, _):
            sl = pl.ds(g * L, L)
            e = alsr[sl] + aldr[sl]
            e = jnp.maximum(e, 0.2 * e)
            p = jnp.exp(e)
            dv = ekr[pl.ds(K + g * L, L)]
            ew = plsc.bitcast(ekr[pl.ds(2 * K + g * L, L)], jnp.float32)
            ge = dv >= NH
            mine = ge == is_hi
            zero = jnp.zeros((L,), jnp.float32)
            c_b[sl] = jnp.where(mine, p * ew, zero)
            pr[sl] = jnp.where(mine, p, zero)
            dlr[sl] = jnp.where(ge, dv - NH, dv)
            return ()

        lax.fori_loop(0, K // L, vec, (), unroll=4)

    def _mrow1(bufs):
        hrowr = bufs[3]
        msgr = bufs[4]

        def mrow(g, _):
            base = g * L
            c16 = c_b[pl.ds(base, L)]
            for l in range(L):
                msgr[base + l] = hrowr[base + l] * c16[l]
            return ()

        lax.fori_loop(0, K // L, mrow, (), unroll=1)

    def _run_pipeline(issue_g, wait_g, vec, mrow, issue_sc, wait_sc):
        _issue_eb(A, 0)

        def body2(i, _):
            a = 2 * i
            b = a + 1
            _wait_eb(A, a)
            issue_g(A)

            @pl.when(i > 0)
            def _():
                wait_g(B)
                vec(B)
                _issue_eb(B, b)
                mrow(B)
                wait_sc(A)
                issue_sc(B)

            @pl.when(i == 0)
            def _():
                _issue_eb(B, b)

            _wait_eb(B, b)
            issue_g(B)
            wait_g(A)
            vec(A)

            @pl.when(i < CHUNKS // 2 - 1)
            def _():
                _issue_eb(A, a + 2)

            mrow(A)

            @pl.when(i > 0)
            def _():
                wait_sc(B)
            issue_sc(A)
            return ()

        lax.fori_loop(0, CHUNKS // 2, body2, ())
        wait_g(B)
        vec(B)
        mrow(B)
        wait_sc(A)
        issue_sc(B)
        wait_sc(B)

    _run_pipeline(_issue_g1, _wait_g1, _vec1, _mrow1, _issue_sc1, _wait_sc1)

    plsc.subcore_barrier()

    iota = lax.iota(jnp.int32, L)
    b1_vec = b1_v[...]
    wt_vec = wt_v[...]

    @pl.when(sid < 10)
    def _mid():
        for t in range(MIDR // 200):
            loc = sid * MIDR + t * 200
            pltpu.sync_copy(acc_sp.at[pl.ds(loc, 200)],
                            msg_a.at[pl.ds(0, 200)])
            pltpu.sync_copy(s_sp.at[pl.ds(loc, 200)], p_a.at[pl.ds(0, 200)])

            def mg(g, _):
                sl = pl.ds(g * L, L)
                ridx = iota + g * L
                inv = 1.0 / (p_a[sl] + EPS)
                acc0 = jnp.zeros((L,), jnp.float32)
                for f in range(16):
                    colv = plsc.load_gather(
                        msg_a, [ridx, jnp.full((L,), f, jnp.int32)])
                    acc0 = acc0 + jnp.maximum(
                        colv * inv + b1_vec[f], 0.0) * wt_vec[f]
                als_a[sl] = acc0
                return ()

            lax.fori_loop(0, 13, mg, ())
            pltpu.sync_copy(als_a.at[pl.ds(0, 200)],
                            ald2_o.at[pl.ds(cid * NH + loc, 200)])

    plsc.subcore_barrier()

    c0v = jnp.zeros((L,), jnp.int32)
    c1v = jnp.full((L,), 1, jnp.int32)
    c2v = jnp.full((L,), 2, jnp.int32)

    def _issue_g2(bufs):
        ekr = bufs[0]
        pltpu.async_copy(rec_h.at[ekr.at[pl.ds(0, K)]], bufs[3], sem_g)
        pltpu.async_copy(ald2_o.at[ekr.at[pl.ds(K, K)]], bufs[2], sem_g)

    def _wait_g2(bufs):
        ekr = bufs[0]
        pltpu.make_async_copy(rec_h.at[ekr.at[pl.ds(0, K)]], bufs[3],
                              sem_g).wait()
        pltpu.make_async_copy(ald2_o.at[ekr.at[pl.ds(K, K)]], bufs[2],
                              sem_g).wait()

    def _issue_sc2(bufs):
        pltpu.async_copy(bufs[7], acc2_sp.at[bufs[6]], sem_s, add=True)

    def _wait_sc2(bufs):
        pltpu.make_async_copy(bufs[7], acc2_sp.at[bufs[6]], sem_s).wait()

    def _vec2(bufs):
        (ekr, _, aldr, hrowr, _, _, dlr, m2r) = bufs

        def vec(g, _):
            sl = pl.ds(g * L, L)
            ridx = iota + g * L
            hv0 = plsc.load_gather(hrowr, [ridx, c0v])
            hv1 = plsc.load_gather(hrowr, [ridx, c1v])
            av = plsc.load_gather(hrowr, [ridx, c2v])
            e = av + aldr[sl]
            e = jnp.maximum(e, 0.2 * e)
            p = jnp.exp(e)
            dv = ekr[pl.ds(K + g * L, L)]
            ew = plsc.bitcast(ekr[pl.ds(2 * K + g * L, L)], jnp.float32)
            ge = dv >= NH
            mine = ge == is_hi
            zero = jnp.zeros((L,), jnp.float32)
            c = jnp.where(mine, p * ew, zero)
            pm = jnp.where(mine, p, zero)
            plsc.store_scatter(m2r, [ridx, c0v], c * hv0)
            plsc.store_scatter(m2r, [ridx, c1v], c * hv1)
            plsc.store_scatter(m2r, [ridx, c2v], pm)
            dlr[sl] = jnp.where(ge, dv - NH, dv)
            return ()

        lax.fori_loop(0, K // L, vec, (), unroll=4)

    def _mrow2(bufs):
        pass

    _run_pipeline(_issue_g2, _wait_g2, _vec2, _mrow2, _issue_sc2, _wait_sc2)

    plsc.subcore_barrier()

    @pl.when(sid < 10)
    def _export():
        for t in range(MIDR // 200):
            loc = sid * MIDR + t * 200
            rs = pl.ds(loc, 200)
            rg = pl.ds(cid * NH + loc, 200)
            pltpu.sync_copy(acc2_sp.at[rs], msg2_a.at[pl.ds(0, 200)])
            pltpu.sync_copy(msg2_a.at[pl.ds(0, 200)], acc2i_o.at[rg])


def _k_bufs():
    return [
        pltpu.VMEM((3 * K,), jnp.int32),
        pltpu.VMEM((K,), jnp.float32),
        pltpu.VMEM((K,), jnp.float32),
        pltpu.VMEM((K, 16), jnp.float32),
        pltpu.VMEM((K, 16), jnp.float32),
        pltpu.VMEM((K,), jnp.float32),
        pltpu.VMEM((K,), jnp.int32),
    ]


_fused_call = pl.kernel(
    _sc_fused,
    out_type=(
        jax.ShapeDtypeStruct((N, 8), jnp.float32),
        jax.ShapeDtypeStruct((N,), jnp.float32),
    ),
    mesh=_mesh,
    compiler_params=_SC_PARAMS,
    scratch_types=[
        pltpu.VMEM_SHARED((NH, 16), jnp.float32),
        pltpu.VMEM_SHARED((NH,), jnp.float32),
        pltpu.VMEM_SHARED((NH, 8), jnp.float32),
        *_k_bufs(),
        *_k_bufs(),
        pltpu.VMEM((K, 8), jnp.float32),
        pltpu.VMEM((K, 8), jnp.float32),
        pltpu.VMEM((K,), jnp.float32),
        pltpu.VMEM((SB,), jnp.float32),
        pltpu.VMEM((16,), jnp.float32),
        pltpu.VMEM((16,), jnp.float32),
        pltpu.SemaphoreType.DMA,
        pltpu.SemaphoreType.DMA,
        pltpu.SemaphoreType.DMA,
    ],
)


_BR = 4000
_DOT = functools.partial(lax.dot_general,
                         dimension_numbers=(((1,), (0,)), ((), ())),
                         preferred_element_type=jnp.float32)


def _tc_proj_body(xu_ref, xb_ref, w1s_ref, w1d_ref, a1s_ref, a1d_ref, w2s_ref,
                  a2s_ref, w2d_ref, a2d_ref,
                  h_ref, als_ref, ald_ref, rec_ref, wt_ref):
    h1 = _DOT(xu_ref[...], w1s_ref[...])
    hd1 = _DOT(xb_ref[...], w1d_ref[...])
    h_ref[...] = h1
    als_ref[...] = jnp.sum(h1 * a1s_ref[...][None, :], axis=-1, keepdims=True)
    ald_ref[...] = jnp.sum(hd1 * a1d_ref[...][None, :], axis=-1, keepdims=True)
    h2 = _DOT(jnp.maximum(h1, 0.0), w2s_ref[...])
    als2 = jnp.sum(h2 * a2s_ref[...][None, :], axis=-1, keepdims=True)
    pad = jnp.zeros((h2.shape[0], 13), jnp.float32)
    rec_ref[...] = jnp.concatenate([h2, als2, pad], axis=-1)
    wt_ref[...] = jnp.sum(w2d_ref[...] * a2d_ref[...][None, :],
                          axis=-1).reshape(1, 16)


def _tc_out_body(ai_ref, b2_ref, o_ref):
    blk = ai_ref[...]
    acc = blk[:, 0:2]
    s = blk[:, 2:3]
    o_ref[...] = acc / (s + EPS) + b2_ref[...][None, :]


def kernel(x_user, x_badge, edge_index, edge_weight,
           W1_src, W1_dst, att1_src, att1_dst, b1,
           W2_src, W2_dst, att2_src, att2_dst, b2):
    src = edge_index[0]
    dst = edge_index[1]

    grid = (N // _BR,)
    full = lambda shp: pl.BlockSpec(shp, lambda i: tuple(0 for _ in shp))
    row2 = lambda w: pl.BlockSpec((_BR, w), lambda i: (i, 0))

    h1, als1, ald1, rec2, wt = pl.pallas_call(
        _tc_proj_body,
        grid=grid,
        in_specs=[row2(128), row2(128), full((128, 16)), full((128, 16)),
                  full((16,)), full((16,)), full((16, 2)), full((2,)),
                  full((16, 2)), full((2,))],
        out_specs=[row2(16), row2(1), row2(1), row2(16), full((1, 16))],
        out_shape=[
            jax.ShapeDtypeStruct((N, 16), jnp.float32),
            jax.ShapeDtypeStruct((N, 1), jnp.float32),
            jax.ShapeDtypeStruct((N, 1), jnp.float32),
            jax.ShapeDtypeStruct((N, 16), jnp.float32),
            jax.ShapeDtypeStruct((1, 16), jnp.float32),
        ],
    )(x_user, x_badge, W1_src, W1_dst, att1_src, att1_dst, W2_src, att2_src,
      W2_dst, att2_dst)

    epk = jnp.concatenate(
        [src.reshape(E // K, K), dst.reshape(E // K, K),
         jax.lax.bitcast_convert_type(edge_weight, jnp.int32).reshape(
             E // K, K)], axis=1)
    acc2i, _ald2 = _fused_call(
        epk, h1, als1.reshape(N), ald1.reshape(N), rec2, b1, wt.reshape(16))

    out = pl.pallas_call(
        _tc_out_body,
        grid=grid,
        in_specs=[row2(8), full((2,))],
        out_specs=row2(2),
        out_shape=jax.ShapeDtypeStruct((N, 2), jnp.float32),
    )(acc2i, b2)
    return out


# speedup vs baseline: 1.0334x; 1.0334x over previous
import functools

import jax
import jax.numpy as jnp
from jax import lax
from jax.experimental import pallas as pl
from jax.experimental.pallas import tpu as pltpu
from jax.experimental.pallas import tpu_sc as plsc

N = 100000
NH = N // 2
E = 1600000
NS = 16
L = 16

K = 400
SB = 2000
EPS = 1e-16
CHUNKS = E // NS // K
STAGE = 10000
MIDR = 5000

_mesh = plsc.VectorSubcoreMesh(core_axis_name="c", subcore_axis_name="s")
_SC_PARAMS = pltpu.CompilerParams(needs_layout_passes=False,
                                  use_tc_tiling_on_sc=False)


def _sc_fused(src_h, dst_h, ew_h, h_hbm, als_h, ald_h, rec_h, b1_h, wt_h,
              acc2i_o, ald2_o,
              acc_sp, s_sp, acc2_sp,
              src_a, dst_a, ew_a, als_a, ald_a, hrow_a, msg_a, p_a, dl_a,
              src_b, dst_b, ew_b, als_b, ald_b, hrow_b, msg_b, p_b, dl_b,
              msg2_a, msg2_b, c_b, sb, b1_v, wt_v,
              sem_e, sem_g, sem_s):
    cid = lax.axis_index("c")
    sid = lax.axis_index("s")
    is_hi = cid == 1

    A = (src_a, dst_a, ew_a, als_a, ald_a, hrow_a, msg_a, p_a, dl_a, msg2_a)
    B = (src_b, dst_b, ew_b, als_b, ald_b, hrow_b, msg_b, p_b, dl_b, msg2_b)

    zv = jnp.zeros((L,), jnp.float32)

    def _zm(j, _):
        msg_a[j] = zv
        return ()

    lax.fori_loop(0, K, _zm, (), unroll=8)

    def _zs(j, _):
        sb[pl.ds(j * L, L)] = zv
        return ()

    lax.fori_loop(0, SB // L, _zs, (), unroll=8)

    @pl.when(sid < 10)
    def _zero_acc():
        for t in range(MIDR // 200):
            rs = pl.ds(sid * MIDR + t * 200, 200)
            pltpu.sync_copy(msg_a.at[pl.ds(0, 200)], acc_sp.at[rs])

    @pl.when(sid < 5)
    def _zero_s():
        for t in range(STAGE // SB):
            rs = pl.ds(sid * STAGE + t * SB, SB)
            pltpu.sync_copy(sb, s_sp.at[rs])

    riota = lax.iota(jnp.int32, L) >> 3
    ciota = lax.iota(jnp.int32, L) & 7
    zv16 = jnp.zeros((L,), jnp.float32)

    def _zm2(j, _):
        rr = riota + 2 * j
        plsc.store_scatter(msg2_a, [rr, ciota], zv16)
        plsc.store_scatter(msg2_b, [rr, ciota], zv16)
        return ()

    lax.fori_loop(0, K // 2, _zm2, (), unroll=8)

    @pl.when(sid < 10)
    def _zero_acc2():
        for t in range(MIDR // 200):
            rs = pl.ds(sid * MIDR + t * 200, 200)
            pltpu.sync_copy(msg2_a.at[pl.ds(0, 200)], acc2_sp.at[rs])

    pltpu.sync_copy(b1_h, b1_v)
    pltpu.sync_copy(wt_h, wt_v)

    plsc.subcore_barrier()

    ebase = sid * (E // NS)

    def _issue_eb(bufs, k):
        es = pl.ds(ebase + k * K, K)
        pltpu.async_copy(src_h.at[es], bufs[0], sem_e)
        pltpu.async_copy(dst_h.at[es], bufs[1], sem_e)
        pltpu.async_copy(ew_h.at[es], bufs[2], sem_e)

    def _wait_eb(bufs, k):
        es = pl.ds(ebase + k * K, K)
        pltpu.make_async_copy(src_h.at[es], bufs[0], sem_e).wait()
        pltpu.make_async_copy(dst_h.at[es], bufs[1], sem_e).wait()
        pltpu.make_async_copy(ew_h.at[es], bufs[2], sem_e).wait()

    def _issue_g1(bufs):
        pltpu.async_copy(als_h.at[bufs[0]], bufs[3], sem_g)
        pltpu.async_copy(ald_h.at[bufs[1]], bufs[4], sem_g)
        pltpu.async_copy(h_hbm.at[bufs[0]], bufs[5], sem_g)

    def _wait_g1(bufs):
        pltpu.make_async_copy(als_h.at[bufs[0]], bufs[3], sem_g).wait()
        pltpu.make_async_copy(ald_h.at[bufs[1]], bufs[4], sem_g).wait()
        pltpu.make_async_copy(h_hbm.at[bufs[0]], bufs[5], sem_g).wait()

    def _issue_sc1(bufs):
        pltpu.async_copy(bufs[6], acc_sp.at[bufs[8]], sem_s, add=True)
        pltpu.async_copy(bufs[7], s_sp.at[bufs[8]], sem_s, add=True)

    def _wait_sc1(bufs):
        pltpu.make_async_copy(bufs[6], acc_sp.at[bufs[8]], sem_s).wait()
        pltpu.make_async_copy(bufs[7], s_sp.at[bufs[8]], sem_s).wait()

    def _vec1(bufs):
        (_, dstr, ewr, alsr, aldr, _, _, pr, dlr, _) = bufs

        def vec(g, _):
            sl = pl.ds(g * L, L)
            e = alsr[sl] + aldr[sl]
            e = jnp.maximum(e, 0.2 * e)
            p = jnp.exp(e)
            dv = dstr[sl]
            ge = dv >= NH
            mine = ge == is_hi
            zero = jnp.zeros((L,), jnp.float32)
            c_b[sl] = jnp.where(mine, p * ewr[sl], zero)
            pr[sl] = jnp.where(mine, p, zero)
            dlr[sl] = jnp.where(ge, dv - NH, dv)
            return ()

        lax.fori_loop(0, K // L, vec, (), unroll=4)

    def _mrow1(bufs):
        hrowr = bufs[5]
        msgr = bufs[6]

        def mrow(g, _):
            base = g * L
            c16 = c_b[pl.ds(base, L)]
            for l in range(L):
                msgr[base + l] = hrowr[base + l] * c16[l]
            return ()

        lax.fori_loop(0, K // L, mrow, (), unroll=1)

    def _run_pipeline(issue_g, wait_g, vec, mrow, issue_sc, wait_sc):
        _issue_eb(A, 0)

        def body2(i, _):
            a = 2 * i
            b = a + 1
            _wait_eb(A, a)
            issue_g(A)

            @pl.when(i > 0)
            def _():
                wait_g(B)
                vec(B)
                _issue_eb(B, b)
                mrow(B)
                wait_sc(A)
                issue_sc(B)

            @pl.when(i == 0)
            def _():
                _issue_eb(B, b)

            _wait_eb(B, b)
            issue_g(B)
            wait_g(A)
            vec(A)

            @pl.when(i < CHUNKS // 2 - 1)
            def _():
                _issue_eb(A, a + 2)

            mrow(A)

            @pl.when(i > 0)
            def _():
                wait_sc(B)
            issue_sc(A)
            return ()

        lax.fori_loop(0, CHUNKS // 2, body2, ())
        wait_g(B)
        vec(B)
        mrow(B)
        wait_sc(A)
        issue_sc(B)
        wait_sc(B)

    _run_pipeline(_issue_g1, _wait_g1, _vec1, _mrow1, _issue_sc1, _wait_sc1)

    plsc.subcore_barrier()

    iota = lax.iota(jnp.int32, L)
    b1_vec = b1_v[...]
    wt_vec = wt_v[...]

    @pl.when(sid < 10)
    def _mid():
        for t in range(MIDR // 200):
            loc = sid * MIDR + t * 200
            pltpu.sync_copy(acc_sp.at[pl.ds(loc, 200)],
                            msg_a.at[pl.ds(0, 200)])
            pltpu.sync_copy(s_sp.at[pl.ds(loc, 200)], p_a.at[pl.ds(0, 200)])

            def mg(g, _):
                sl = pl.ds(g * L, L)
                ridx = iota + g * L
                inv = 1.0 / (p_a[sl] + EPS)
                acc0 = jnp.zeros((L,), jnp.float32)
                for f in range(16):
                    colv = plsc.load_gather(
                        msg_a, [ridx, jnp.full((L,), f, jnp.int32)])
                    acc0 = acc0 + jnp.maximum(
                        colv * inv + b1_vec[f], 0.0) * wt_vec[f]
                als_a[sl] = acc0
                return ()

            lax.fori_loop(0, 13, mg, ())
            pltpu.sync_copy(als_a.at[pl.ds(0, 200)],
                            ald2_o.at[pl.ds(cid * NH + loc, 200)])

    plsc.subcore_barrier()

    c0v = jnp.zeros((L,), jnp.int32)
    c1v = jnp.full((L,), 1, jnp.int32)
    c2v = jnp.full((L,), 2, jnp.int32)

    def _issue_g2(bufs):
        pltpu.async_copy(rec_h.at[bufs[0]], bufs[5], sem_g)
        pltpu.async_copy(ald2_o.at[bufs[1]], bufs[4], sem_g)

    def _wait_g2(bufs):
        pltpu.make_async_copy(rec_h.at[bufs[0]], bufs[5], sem_g).wait()
        pltpu.make_async_copy(ald2_o.at[bufs[1]], bufs[4], sem_g).wait()

    def _issue_sc2(bufs):
        pltpu.async_copy(bufs[9], acc2_sp.at[bufs[8]], sem_s, add=True)

    def _wait_sc2(bufs):
        pltpu.make_async_copy(bufs[9], acc2_sp.at[bufs[8]], sem_s).wait()

    def _vec2(bufs):
        (_, dstr, ewr, _, aldr, hrowr, _, _, dlr, m2r) = bufs

        def vec(g, _):
            sl = pl.ds(g * L, L)
            ridx = iota + g * L
            hv0 = plsc.load_gather(hrowr, [ridx, c0v])
            hv1 = plsc.load_gather(hrowr, [ridx, c1v])
            av = plsc.load_gather(hrowr, [ridx, c2v])
            e = av + aldr[sl]
            e = jnp.maximum(e, 0.2 * e)
            p = jnp.exp(e)
            dv = dstr[sl]
            ge = dv >= NH
            mine = ge == is_hi
            zero = jnp.zeros((L,), jnp.float32)
            c = jnp.where(mine, p * ewr[sl], zero)
            pm = jnp.where(mine, p, zero)
            plsc.store_scatter(m2r, [ridx, c0v], c * hv0)
            plsc.store_scatter(m2r, [ridx, c1v], c * hv1)
            plsc.store_scatter(m2r, [ridx, c2v], pm)
            dlr[sl] = jnp.where(ge, dv - NH, dv)
            return ()

        lax.fori_loop(0, K // L, vec, (), unroll=4)

    def _mrow2(bufs):
        pass

    _run_pipeline(_issue_g2, _wait_g2, _vec2, _mrow2, _issue_sc2, _wait_sc2)

    plsc.subcore_barrier()

    @pl.when(sid < 10)
    def _export():
        for t in range(MIDR // 200):
            loc = sid * MIDR + t * 200
            rs = pl.ds(loc, 200)
            rg = pl.ds(cid * NH + loc, 200)
            pltpu.sync_copy(acc2_sp.at[rs], msg2_a.at[pl.ds(0, 200)])
            pltpu.sync_copy(msg2_a.at[pl.ds(0, 200)], acc2i_o.at[rg])


def _k_bufs():
    return [
        pltpu.VMEM((K,), jnp.int32),
        pltpu.VMEM((K,), jnp.int32),
        pltpu.VMEM((K,), jnp.float32),
        pltpu.VMEM((K,), jnp.float32),
        pltpu.VMEM((K,), jnp.float32),
        pltpu.VMEM((K, 16), jnp.float32),
        pltpu.VMEM((K, 16), jnp.float32),
        pltpu.VMEM((K,), jnp.float32),
        pltpu.VMEM((K,), jnp.int32),
    ]


_fused_call = pl.kernel(
    _sc_fused,
    out_type=(
        jax.ShapeDtypeStruct((N, 8), jnp.float32),
        jax.ShapeDtypeStruct((N,), jnp.float32),
    ),
    mesh=_mesh,
    compiler_params=_SC_PARAMS,
    scratch_types=[
        pltpu.VMEM_SHARED((NH, 16), jnp.float32),
        pltpu.VMEM_SHARED((NH,), jnp.float32),
        pltpu.VMEM_SHARED((NH, 8), jnp.float32),
        *_k_bufs(),
        *_k_bufs(),
        pltpu.VMEM((K, 8), jnp.float32),
        pltpu.VMEM((K, 8), jnp.float32),
        pltpu.VMEM((K,), jnp.float32),
        pltpu.VMEM((SB,), jnp.float32),
        pltpu.VMEM((16,), jnp.float32),
        pltpu.VMEM((16,), jnp.float32),
        pltpu.SemaphoreType.DMA,
        pltpu.SemaphoreType.DMA,
        pltpu.SemaphoreType.DMA,
    ],
)


_BR = 4000
_DOT = functools.partial(lax.dot_general,
                         dimension_numbers=(((1,), (0,)), ((), ())),
                         preferred_element_type=jnp.float32)


def _tc_proj_body(xu_ref, xb_ref, w1s_ref, w1d_ref, a1s_ref, a1d_ref, w2s_ref,
                  a2s_ref, w2d_ref, a2d_ref,
                  h_ref, als_ref, ald_ref, rec_ref, wt_ref):
    h1 = _DOT(xu_ref[...], w1s_ref[...])
    hd1 = _DOT(xb_ref[...], w1d_ref[...])
    h_ref[...] = h1
    als_ref[...] = jnp.sum(h1 * a1s_ref[...][None, :], axis=-1, keepdims=True)
    ald_ref[...] = jnp.sum(hd1 * a1d_ref[...][None, :], axis=-1, keepdims=True)
    h2 = _DOT(jnp.maximum(h1, 0.0), w2s_ref[...])
    als2 = jnp.sum(h2 * a2s_ref[...][None, :], axis=-1, keepdims=True)
    pad = jnp.zeros((h2.shape[0], 13), jnp.float32)
    rec_ref[...] = jnp.concatenate([h2, als2, pad], axis=-1)
    wt_ref[...] = jnp.sum(w2d_ref[...] * a2d_ref[...][None, :],
                          axis=-1).reshape(1, 16)


def _tc_out_body(ai_ref, b2_ref, o_ref):
    blk = ai_ref[...]
    acc = blk[:, 0:2]
    s = blk[:, 2:3]
    o_ref[...] = acc / (s + EPS) + b2_ref[...][None, :]


def kernel(x_user, x_badge, edge_index, edge_weight,
           W1_src, W1_dst, att1_src, att1_dst, b1,
           W2_src, W2_dst, att2_src, att2_dst, b2):
    src = edge_index[0]
    dst = edge_index[1]

    grid = (N // _BR,)
    full = lambda shp: pl.BlockSpec(shp, lambda i: tuple(0 for _ in shp))
    row2 = lambda w: pl.BlockSpec((_BR, w), lambda i: (i, 0))

    h1, als1, ald1, rec2, wt = pl.pallas_call(
        _tc_proj_body,
        grid=grid,
        in_specs=[row2(128), row2(128), full((128, 16)), full((128, 16)),
                  full((16,)), full((16,)), full((16, 2)), full((2,)),
                  full((16, 2)), full((2,))],
        out_specs=[row2(16), row2(1), row2(1), row2(16), full((1, 16))],
        out_shape=[
            jax.ShapeDtypeStruct((N, 16), jnp.float32),
            jax.ShapeDtypeStruct((N, 1), jnp.float32),
            jax.ShapeDtypeStruct((N, 1), jnp.float32),
            jax.ShapeDtypeStruct((N, 16), jnp.float32),
            jax.ShapeDtypeStruct((1, 16), jnp.float32),
        ],
    )(x_user, x_badge, W1_src, W1_dst, att1_src, att1_dst, W2_src, att2_src,
      W2_dst, att2_dst)

    acc2i, _ald2 = _fused_call(
        src, dst, edge_weight, h1,
        als1.reshape(N), ald1.reshape(N), rec2, b1, wt.reshape(16))

    out = pl.pallas_call(
        _tc_out_body,
        grid=grid,
        in_specs=[row2(8), full((2,))],
        out_specs=row2(2),
        out_shape=jax.ShapeDtypeStruct((N, 2), jnp.float32),
    )(acc2i, b2)
    return out
